# baseline (speedup 1.0000x reference)
import functools

import jax
import jax.numpy as jnp
from jax import lax
from jax.experimental import pallas as pl
from jax.experimental.pallas import tpu as pltpu
from jax.experimental.pallas import tpu_sc as plsc

B = 1024
L = 200
E = 128
NC = 2
NS = 16
NW = NC * NS
ROWS_PER_W = B // NW
CH = 2
CL = L // CH

_mesh = plsc.VectorSubcoreMesh(core_axis_name="c", subcore_axis_name="s")


@functools.partial(
    pl.kernel,
    mesh=_mesh,
    out_type=jax.ShapeDtypeStruct((B, L, E), jnp.float32),
    scratch_types=[
        pltpu.VMEM((ROWS_PER_W, CH, CL), jnp.int32),
        pltpu.VMEM((L, E), jnp.float32),
        pltpu.VMEM((2, L, E), jnp.float32),
        pltpu.SemaphoreType.DMA,
        pltpu.SemaphoreType.DMA,
        pltpu.SemaphoreType.DMA,
        pltpu.SemaphoreType.DMA,
    ],
)
def _emb_kernel(x_hbm, tok_hbm, pos_hbm, out_hbm, idx_v, pos_v, rows_v,
                gsem0, gsem1, osem0, osem1):
    wid = lax.axis_index("s") * NC + lax.axis_index("c")
    base = wid * ROWS_PER_W
    pltpu.sync_copy(x_hbm.at[pl.ds(base, ROWS_PER_W)], idx_v)
    pltpu.sync_copy(pos_hbm, pos_v)

    gsems = (gsem0, gsem1)
    osems = (osem0, osem1)

    def add_rows(b):
        def add_body(r, _):
            for j in range(E // 16):
                sl = pl.ds(j * 16, 16)
                rows_v[b, r, sl] = rows_v[b, r, sl] + pos_v[r, sl]
            return 0
        lax.fori_loop(0, L, add_body, 0)

    gcp = [None, None]
    ocp = [None, None]
    for i in range(ROWS_PER_W):
        b = i % 2
        if ocp[b] is not None:
            ocp[b].wait()
        gcp[b] = [
            pltpu.async_copy(
                tok_hbm.at[idx_v.at[i, c]],
                rows_v.at[b, pl.ds(c * CL, CL)],
                gsems[b],
            )
            for c in range(CH)
        ]
        if i >= 1:
            pb = (i - 1) % 2
            for cp in gcp[pb]:
                cp.wait()
            add_rows(pb)
            ocp[pb] = pltpu.async_copy(
                rows_v.at[pb], out_hbm.at[base + i - 1], osems[pb]
            )
    lb = (ROWS_PER_W - 1) % 2
    for cp in gcp[lb]:
        cp.wait()
    add_rows(lb)
    ocp[lb] = pltpu.async_copy(
        rows_v.at[lb], out_hbm.at[base + ROWS_PER_W - 1], osems[lb]
    )
    for b in range(2):
        ocp[b].wait()


CHK = 8
NCHK = L // CHK
NBUF = 3


def _mask_body(xt_ref, o_ref, mbuf, buf, sem):
    mbuf[...] = (xt_ref[...] != 0).astype(jnp.int32)

    def body(k, _):
        bsel = lax.rem(k, NBUF)

        @pl.when(k >= NBUF)
        def _drain():
            pltpu.make_async_copy(
                buf.at[0], o_ref.at[pl.ds(0, CHK)], sem
            ).wait()

        mi = mbuf[pl.ds(k * CHK, CHK), :]
        buf[bsel] = mi[:, None, :] & mbuf[...][None, :, :]
        pltpu.make_async_copy(
            buf.at[bsel], o_ref.at[pl.ds(k * CHK, CHK)], sem
        ).start()
        return 0

    lax.fori_loop(0, NCHK, body, 0)
    for _ in range(NBUF):
        pltpu.make_async_copy(buf.at[0], o_ref.at[pl.ds(0, CHK)], sem).wait()


def kernel(x, token_table, pos_table):
    x_sc = x.reshape(B, CH, CL)
    x_t = x.T
    mask_t = pl.pallas_call(
        _mask_body,
        in_specs=[pl.BlockSpec(memory_space=pltpu.VMEM)],
        out_specs=pl.BlockSpec(memory_space=pl.ANY),
        out_shape=jax.ShapeDtypeStruct((L, L, B), jnp.int32),
        scratch_shapes=[
            pltpu.VMEM((L, B), jnp.int32),
            pltpu.VMEM((NBUF, CHK, L, B), jnp.int32),
            pltpu.SemaphoreType.DMA,
        ],
    )(x_t)
    attn_mask = jnp.transpose(mask_t, (2, 0, 1)).reshape(B, 1, L, L)
    out = _emb_kernel(x_sc, token_table, pos_table)
    return out, attn_mask


# speedup vs baseline: 1.0389x; 1.0389x over previous
import functools

import jax
import jax.numpy as jnp
from jax import lax
from jax.experimental import pallas as pl
from jax.experimental.pallas import tpu as pltpu
from jax.experimental.pallas import tpu_sc as plsc

B = 1024
L = 200
E = 128
NC = 2
NS = 16
NW = NC * NS
ROWS_PER_W = B // NW
CH = 2
CL = L // CH
NB = 3

_mesh = plsc.VectorSubcoreMesh(core_axis_name="c", subcore_axis_name="s")


@functools.partial(
    pl.kernel,
    mesh=_mesh,
    out_type=jax.ShapeDtypeStruct((B, L, E), jnp.float32),
    scratch_types=[
        pltpu.VMEM((ROWS_PER_W, CH, CL), jnp.int32),
        pltpu.VMEM((L, E), jnp.float32),
        pltpu.VMEM((NB, L, E), jnp.float32),
        pltpu.SemaphoreType.DMA,
        pltpu.SemaphoreType.DMA,
        pltpu.SemaphoreType.DMA,
        pltpu.SemaphoreType.DMA,
        pltpu.SemaphoreType.DMA,
        pltpu.SemaphoreType.DMA,
        pltpu.SemaphoreType.DMA,
        pltpu.SemaphoreType.DMA,
    ],
)
def _emb_kernel(x_hbm, tok_hbm, pos_hbm, out_hbm, idx_v, pos_v, rows_v,
                gsem0, gsem1, gsem2, osem0, osem1, osem2, isem, psem):
    wid = lax.axis_index("s") * NC + lax.axis_index("c")
    base = wid * ROWS_PER_W
    gsems = (gsem0, gsem1, gsem2)
    osems = (osem0, osem1, osem2)

    icp = pltpu.async_copy(x_hbm.at[pl.ds(base, ROWS_PER_W)], idx_v, isem)
    pcp = pltpu.async_copy(pos_hbm, pos_v, psem)
    icp.wait()

    def add_rows(b):
        def add_body(r, _):
            for j in range(E // 16):
                sl = pl.ds(j * 16, 16)
                rows_v[b, r, sl] = rows_v[b, r, sl] + pos_v[r, sl]
            return 0
        lax.fori_loop(0, L, add_body, 0)

    gcp = [None] * NB
    ocp = [None] * NB
    for i in range(ROWS_PER_W):
        b = i % NB
        if ocp[b] is not None:
            ocp[b].wait()
        gcp[b] = [
            pltpu.async_copy(
                tok_hbm.at[idx_v.at[i, c]],
                rows_v.at[b, pl.ds(c * CL, CL)],
                gsems[b],
            )
            for c in range(CH)
        ]
        if i == 0:
            pcp.wait()
        if i >= 1:
            pb = (i - 1) % NB
            for cp in gcp[pb]:
                cp.wait()
            add_rows(pb)
            ocp[pb] = pltpu.async_copy(
                rows_v.at[pb], out_hbm.at[base + i - 1], osems[pb]
            )
    lb = (ROWS_PER_W - 1) % NB
    for cp in gcp[lb]:
        cp.wait()
    add_rows(lb)
    ocp[lb] = pltpu.async_copy(
        rows_v.at[lb], out_hbm.at[base + ROWS_PER_W - 1], osems[lb]
    )
    for b in range(NB):
        if ocp[b] is not None:
            ocp[b].wait()


IB = 8


def _mask_body(xi_ref, xall_ref, o_ref):
    mi = xi_ref[...] != 0
    mj = xall_ref[...] != 0
    both = mi[:, None, :] & mj[None, :, :]
    o_ref[...] = both.astype(jnp.int32)


def kernel(x, token_table, pos_table):
    x_sc = x.reshape(B, CH, CL)
    x_t = x.T
    mask_t = pl.pallas_call(
        _mask_body,
        grid=(L // IB,),
        in_specs=[
            pl.BlockSpec((IB, B), lambda i: (i, 0)),
            pl.BlockSpec((L, B), lambda i: (0, 0)),
        ],
        out_specs=pl.BlockSpec((IB, L, B), lambda i: (i, 0, 0)),
        out_shape=jax.ShapeDtypeStruct((L, L, B), jnp.int32),
    )(x_t, x_t)
    attn_mask = jnp.transpose(mask_t, (2, 0, 1)).reshape(B, 1, L, L)
    out = _emb_kernel(x_sc, token_table, pos_table)
    return out, attn_mask
